# baseline (speedup 1.0000x reference)
import functools

import jax
import jax.numpy as jnp
from jax import lax
from jax.experimental import pallas as pl
from jax.experimental.pallas import tpu as pltpu
from jax.experimental.pallas import tpu_sc as plsc

_K = 20
_R = 512
_CD = 32
_GSUB = 80


def _tc_layer_body(din, dout, nblk, refs):
    xf_ref, w_ref, b_ref, idx_ref, upre_ref, v_ref = refs
    m_total = nblk * _R
    i = pl.program_id(0)

    @pl.when(i == 0)
    def _():
        v = jnp.dot(xf_ref[...], w_ref[din:, :],
                    preferred_element_type=jnp.float32)
        if dout < 128:
            v = jnp.concatenate(
                [v, jnp.zeros((m_total, 128 - dout), jnp.float32)], axis=1)
        v_ref[...] = v

    xb = xf_ref[pl.ds(i * _R, _R), :]
    xf = xf_ref[...]
    ones = jnp.ones((8, din), jnp.float32)
    d2 = lax.dot_general(ones, xf * xf, (((1,), (1,)), ((), ())),
                         preferred_element_type=jnp.float32)[:1, :]
    xdot = lax.dot_general(xb, xf, (((1,), (1,)), ((), ())),
                           preferred_element_type=jnp.float32)
    dist = d2 - 2.0 * xdot
    col = lax.broadcasted_iota(jnp.int32, (_R, m_total), 1)
    row = i * _R + lax.broadcasted_iota(jnp.int32, (_R, m_total), 0)
    dist = jnp.where(col == row, dist + 1e10, dist)

    picks = []
    for _ in range(_K):
        idxv = jnp.argmin(dist, axis=1).astype(jnp.int32)[:, None]
        dist = jnp.where(col == idxv, jnp.inf, dist)
        picks.append(idxv)
    idx_ref[...] = jnp.concatenate(picks, axis=1)

    u = jnp.dot(xb, w_ref[:din, :] - w_ref[din:, :],
                preferred_element_type=jnp.float32)
    upre_ref[...] = u + b_ref[...]


def _tc_layer(xf, w, b):
    m, din = xf.shape
    dout = w.shape[1]
    nblk = m // _R
    full = lambda s: pl.BlockSpec(s, lambda i: (0, 0))
    body = lambda *refs: _tc_layer_body(din, dout, nblk, refs)
    return pl.pallas_call(
        body,
        grid=(nblk,),
        in_specs=[full((m, din)), full((2 * din, dout)), full((1, dout))],
        out_specs=(pl.BlockSpec((_R, _K), lambda i: (i, 0)),
                   pl.BlockSpec((_R, dout), lambda i: (i, 0)),
                   full((m, 128))),
        out_shape=(jax.ShapeDtypeStruct((m, _K), jnp.int32),
                   jax.ShapeDtypeStruct((m, dout), jnp.float32),
                   jax.ShapeDtypeStruct((m, 128), jnp.float32)),
        scratch_shapes=[],
    )(xf, w, b.reshape(1, dout))


def _sc_gather_max(v, upre, idx_flat):
    m, dout = upre.shape
    info = plsc.get_sparse_core_info()
    nw = info.num_cores * info.num_subcores
    rows_w = m // nw
    nchunk = rows_w // _CD
    mesh = plsc.VectorSubcoreMesh(core_axis_name="c", subcore_axis_name="s")

    @functools.partial(
        pl.kernel, mesh=mesh,
        out_type=jax.ShapeDtypeStruct((m, dout), jnp.float32),
        scratch_types=[
            pltpu.VMEM((_CD * _K,), jnp.int32),
            pltpu.VMEM((_CD * _K, 128), jnp.float32),
            pltpu.VMEM((_CD, dout), jnp.float32),
            pltpu.VMEM((_CD, dout), jnp.float32),
            pltpu.SemaphoreType.DMA,
        ],
    )
    def k(v_hbm, upre_hbm, idx_hbm, out_hbm, idx_v, rows_v, u_v, out_v, sem):
        wid = lax.axis_index("s") * info.num_cores + lax.axis_index("c")
        row0 = wid * rows_w

        def chunk(c, _):
            base = row0 + c * _CD
            pltpu.sync_copy(idx_hbm.at[pl.ds(base * _K, _CD * _K)], idx_v)
            copies = [
                pltpu.async_copy(
                    v_hbm.at[idx_v.at[pl.ds(s * _GSUB, _GSUB)]],
                    rows_v.at[pl.ds(s * _GSUB, _GSUB), :], sem)
                for s in range((_CD * _K) // _GSUB)
            ]
            pltpu.sync_copy(upre_hbm.at[pl.ds(base, _CD), :], u_v)
            for cp in copies:
                cp.wait()

            def one_row(r, _):
                for q in range(dout // 16):
                    fs = pl.ds(q * 16, 16)
                    acc = rows_v[r * _K, fs]
                    for t in range(1, _K):
                        acc = jnp.maximum(acc, rows_v[r * _K + t, fs])
                    out_v[r, fs] = jnp.maximum(acc + u_v[r, fs], 0.0)
                return _

            lax.fori_loop(0, _CD, one_row, 0, unroll=False)
            pltpu.sync_copy(out_v, out_hbm.at[pl.ds(base, _CD), :])
            return _

        lax.fori_loop(0, nchunk, chunk, 0, unroll=False)

    return k(v, upre, idx_flat)


def _head_body(refs):
    xf_ref, wf1_ref, bf1_ref, wf2_ref, bf2_ref, out_ref = refs
    h = jnp.maximum(jnp.dot(xf_ref[...], wf1_ref[...],
                            preferred_element_type=jnp.float32)
                    + bf1_ref[...], 0.0)
    out_ref[...] = jnp.dot(h, wf2_ref[...],
                           preferred_element_type=jnp.float32) + bf2_ref[...]


def _head(xf, wf1, bf1, wf2, bf2):
    m, d = xf.shape
    full = lambda s: pl.BlockSpec(s, lambda: (0, 0))
    return pl.pallas_call(
        lambda *refs: _head_body(refs),
        in_specs=[full((m, d)), full(wf1.shape), full((1, wf1.shape[1])),
                  full(wf2.shape), full((1, wf2.shape[1]))],
        out_specs=full((m, wf2.shape[1])),
        out_shape=jax.ShapeDtypeStruct((m, wf2.shape[1]), jnp.float32),
    )(xf, wf1, bf1.reshape(1, -1), wf2, bf2.reshape(1, -1))


def kernel(x, W1, b1, W2, b2, W3, b3, Wf1, bf1, Wf2, bf2):
    B, N, _ = x.shape
    xf = x.reshape(B * N, 3)
    for w, b in ((W1, b1), (W2, b2), (W3, b3)):
        idx, upre, v = _tc_layer(xf, w, b)
        xf = _sc_gather_max(v, upre, idx.reshape(-1))
    out = _head(xf, Wf1, bf1, Wf2, bf2)
    return out.reshape(B, N, 1)


# speedup vs baseline: 1.0774x; 1.0774x over previous
import functools

import jax
import jax.numpy as jnp
from jax import lax
from jax.experimental import pallas as pl
from jax.experimental.pallas import tpu as pltpu
from jax.experimental.pallas import tpu_sc as plsc

_K = 20
_R = 128
_CD = 32
_GSUB = 80


def _tc_layer_body(din, dout, nblk, refs):
    xf_ref, w_ref, b_ref, idx_ref, upre_ref, v_ref = refs
    m_total = nblk * _R
    i = pl.program_id(0)

    @pl.when(i == 0)
    def _():
        v = jnp.dot(xf_ref[...], w_ref[din:, :],
                    preferred_element_type=jnp.float32)
        if dout < 128:
            v = jnp.concatenate(
                [v, jnp.zeros((m_total, 128 - dout), jnp.float32)], axis=1)
        v_ref[...] = v

    xb = xf_ref[pl.ds(i * _R, _R), :]
    xf = xf_ref[...]
    ones = jnp.ones((8, din), jnp.float32)
    d2 = lax.dot_general(ones, xf * xf, (((1,), (1,)), ((), ())),
                         preferred_element_type=jnp.float32)[:1, :]
    xdot = lax.dot_general(xb, xf, (((1,), (1,)), ((), ())),
                           preferred_element_type=jnp.float32)
    dist = d2 - 2.0 * xdot
    col = lax.broadcasted_iota(jnp.int32, (_R, m_total), 1)
    row = i * _R + lax.broadcasted_iota(jnp.int32, (_R, m_total), 0)
    dist = jnp.where(col == row, dist + 1e10, dist)

    ns = m_total // 128
    strips = [dist[:, s * 128:(s + 1) * 128] for s in range(ns)]
    cols = [col[:, s * 128:(s + 1) * 128] for s in range(ns)]
    picks = []
    prev = None
    for t in range(_K):
        bv = None
        nxt = []
        for s in range(ns):
            d = strips[s]
            if prev is not None:
                d = jnp.where(cols[s] == prev, jnp.inf, d)
            nxt.append(d)
            if bv is None:
                bv, bi = d, cols[s]
            else:
                better = d < bv
                bv = jnp.where(better, d, bv)
                bi = jnp.where(better, cols[s], bi)
        strips = nxt
        m = jnp.min(bv, axis=1, keepdims=True)
        prev = jnp.min(jnp.where(bv == m, bi, m_total), axis=1, keepdims=True)
        picks.append(prev)
    idx_ref[...] = jnp.concatenate(picks, axis=1)

    u = jnp.dot(xb, w_ref[:din, :] - w_ref[din:, :],
                preferred_element_type=jnp.float32)
    upre_ref[...] = u + b_ref[...]


def _tc_layer(xf, w, b):
    m, din = xf.shape
    dout = w.shape[1]
    nblk = m // _R
    full = lambda s: pl.BlockSpec(s, lambda i: (0, 0))
    body = lambda *refs: _tc_layer_body(din, dout, nblk, refs)
    return pl.pallas_call(
        body,
        grid=(nblk,),
        in_specs=[full((m, din)), full((2 * din, dout)), full((1, dout))],
        out_specs=(pl.BlockSpec((_R, _K), lambda i: (i, 0)),
                   pl.BlockSpec((_R, dout), lambda i: (i, 0)),
                   full((m, 128))),
        out_shape=(jax.ShapeDtypeStruct((m, _K), jnp.int32),
                   jax.ShapeDtypeStruct((m, dout), jnp.float32),
                   jax.ShapeDtypeStruct((m, 128), jnp.float32)),
        scratch_shapes=[],
    )(xf, w, b.reshape(1, dout))


def _sc_gather_max(v, upre, idx_flat):
    m, dout = upre.shape
    info = plsc.get_sparse_core_info()
    nw = info.num_cores * info.num_subcores
    rows_w = m // nw
    nchunk = rows_w // _CD
    mesh = plsc.VectorSubcoreMesh(core_axis_name="c", subcore_axis_name="s")

    @functools.partial(
        pl.kernel, mesh=mesh,
        out_type=jax.ShapeDtypeStruct((m, dout), jnp.float32),
        scratch_types=[
            pltpu.VMEM((_CD * _K,), jnp.int32),
            pltpu.VMEM((_CD * _K, 128), jnp.float32),
            pltpu.VMEM((_CD, dout), jnp.float32),
            pltpu.VMEM((_CD, dout), jnp.float32),
            pltpu.SemaphoreType.DMA,
        ],
    )
    def k(v_hbm, upre_hbm, idx_hbm, out_hbm, idx_v, rows_v, u_v, out_v, sem):
        wid = lax.axis_index("s") * info.num_cores + lax.axis_index("c")
        row0 = wid * rows_w

        def chunk(c, _):
            base = row0 + c * _CD
            pltpu.sync_copy(idx_hbm.at[pl.ds(base * _K, _CD * _K)], idx_v)
            copies = [
                pltpu.async_copy(
                    v_hbm.at[idx_v.at[pl.ds(s * _GSUB, _GSUB)]],
                    rows_v.at[pl.ds(s * _GSUB, _GSUB), :], sem)
                for s in range((_CD * _K) // _GSUB)
            ]
            pltpu.sync_copy(upre_hbm.at[pl.ds(base, _CD), :], u_v)
            for cp in copies:
                cp.wait()

            def one_row(r, _):
                for q in range(dout // 16):
                    fs = pl.ds(q * 16, 16)
                    acc = rows_v[r * _K, fs]
                    for t in range(1, _K):
                        acc = jnp.maximum(acc, rows_v[r * _K + t, fs])
                    out_v[r, fs] = jnp.maximum(acc + u_v[r, fs], 0.0)
                return _

            lax.fori_loop(0, _CD, one_row, 0, unroll=False)
            pltpu.sync_copy(out_v, out_hbm.at[pl.ds(base, _CD), :])
            return _

        lax.fori_loop(0, nchunk, chunk, 0, unroll=False)

    return k(v, upre, idx_flat)


def _head_body(refs):
    xf_ref, wf1_ref, bf1_ref, wf2_ref, bf2_ref, out_ref = refs
    h = jnp.maximum(jnp.dot(xf_ref[...], wf1_ref[...],
                            preferred_element_type=jnp.float32)
                    + bf1_ref[...], 0.0)
    out_ref[...] = jnp.dot(h, wf2_ref[...],
                           preferred_element_type=jnp.float32) + bf2_ref[...]


def _head(xf, wf1, bf1, wf2, bf2):
    m, d = xf.shape
    full = lambda s: pl.BlockSpec(s, lambda: (0, 0))
    return pl.pallas_call(
        lambda *refs: _head_body(refs),
        in_specs=[full((m, d)), full(wf1.shape), full((1, wf1.shape[1])),
                  full(wf2.shape), full((1, wf2.shape[1]))],
        out_specs=full((m, wf2.shape[1])),
        out_shape=jax.ShapeDtypeStruct((m, wf2.shape[1]), jnp.float32),
    )(xf, wf1, bf1.reshape(1, -1), wf2, bf2.reshape(1, -1))


def kernel(x, W1, b1, W2, b2, W3, b3, Wf1, bf1, Wf2, bf2):
    B, N, _ = x.shape
    xf = x.reshape(B * N, 3)
    for w, b in ((W1, b1), (W2, b2), (W3, b3)):
        idx, upre, v = _tc_layer(xf, w, b)
        xf = _sc_gather_max(v, upre, idx.reshape(-1))
    out = _head(xf, Wf1, bf1, Wf2, bf2)
    return out.reshape(B, N, 1)


# speedup vs baseline: 1.2362x; 1.1474x over previous
import functools

import jax
import jax.numpy as jnp
from jax import lax
from jax.experimental import pallas as pl
from jax.experimental.pallas import tpu as pltpu
from jax.experimental.pallas import tpu_sc as plsc

_K = 20
_R = 256
_CD = 32
_GSUB = 80


def _tc_layer_body(din, dout, nblk, refs):
    xf_ref, w_ref, b_ref, idx_ref, upre_ref, v_ref, d2c_ref, d2r_ref = refs
    m_total = nblk * _R
    i = pl.program_id(0)

    @pl.when(i == 0)
    def _():
        xf0 = xf_ref[...]
        v = jnp.dot(xf0, w_ref[din:, :],
                    preferred_element_type=jnp.float32)
        if dout < 128:
            v = jnp.concatenate(
                [v, jnp.zeros((m_total, 128 - dout), jnp.float32)], axis=1)
        v_ref[...] = v
        d2 = jnp.sum(xf0 * xf0, axis=1, keepdims=True)
        d2c_ref[...] = d2
        d2r_ref[...] = lax.transpose(d2, (1, 0))

    xb = xf_ref[pl.ds(i * _R, _R), :]
    xf = xf_ref[...]
    xdot = lax.dot_general(xb, xf, (((1,), (1,)), ((), ())),
                           preferred_element_type=jnp.float32)
    dist = (d2c_ref[pl.ds(i * _R, _R), :] + d2r_ref[...]) - 2.0 * xdot
    col = lax.broadcasted_iota(jnp.int32, (_R, m_total), 1)
    row = i * _R + lax.broadcasted_iota(jnp.int32, (_R, m_total), 0)
    dist = jnp.where(col == row, dist + 1e10, dist)

    picks = []
    for _ in range(_K):
        idxv = jnp.argmin(dist, axis=1).astype(jnp.int32)[:, None]
        dist = jnp.where(col == idxv, jnp.inf, dist)
        picks.append(idxv)
    idx_ref[...] = jnp.concatenate(picks, axis=1)

    u = jnp.dot(xb, w_ref[:din, :] - w_ref[din:, :],
                preferred_element_type=jnp.float32)
    upre_ref[...] = u + b_ref[...]


def _tc_layer(xf, w, b):
    m, din = xf.shape
    dout = w.shape[1]
    nblk = m // _R
    full = lambda s: pl.BlockSpec(s, lambda i: (0, 0))
    body = lambda *refs: _tc_layer_body(din, dout, nblk, refs)
    return pl.pallas_call(
        body,
        grid=(nblk,),
        in_specs=[full((m, din)), full((2 * din, dout)), full((1, dout))],
        out_specs=(pl.BlockSpec((_R, _K), lambda i: (i, 0)),
                   pl.BlockSpec((_R, dout), lambda i: (i, 0)),
                   full((m, 128))),
        out_shape=(jax.ShapeDtypeStruct((m, _K), jnp.int32),
                   jax.ShapeDtypeStruct((m, dout), jnp.float32),
                   jax.ShapeDtypeStruct((m, 128), jnp.float32)),
        scratch_shapes=[pltpu.VMEM((m, 1), jnp.float32),
                        pltpu.VMEM((1, m), jnp.float32)],
    )(xf, w, b.reshape(1, dout))


def _sc_gather_max(v, upre, idx_flat):
    m, dout = upre.shape
    info = plsc.get_sparse_core_info()
    nw = info.num_cores * info.num_subcores
    rows_w = m // nw
    nchunk = rows_w // _CD
    mesh = plsc.VectorSubcoreMesh(core_axis_name="c", subcore_axis_name="s")

    @functools.partial(
        pl.kernel, mesh=mesh,
        out_type=jax.ShapeDtypeStruct((m, dout), jnp.float32),
        scratch_types=[
            pltpu.VMEM((_CD * _K,), jnp.int32),
            pltpu.VMEM((_CD * _K, 128), jnp.float32),
            pltpu.VMEM((_CD, dout), jnp.float32),
            pltpu.VMEM((_CD, dout), jnp.float32),
            pltpu.SemaphoreType.DMA,
        ],
    )
    def k(v_hbm, upre_hbm, idx_hbm, out_hbm, idx_v, rows_v, u_v, out_v, sem):
        wid = lax.axis_index("s") * info.num_cores + lax.axis_index("c")
        row0 = wid * rows_w

        def chunk(c, _):
            base = row0 + c * _CD
            pltpu.sync_copy(idx_hbm.at[pl.ds(base * _K, _CD * _K)], idx_v)
            copies = [
                pltpu.async_copy(
                    v_hbm.at[idx_v.at[pl.ds(s * _GSUB, _GSUB)]],
                    rows_v.at[pl.ds(s * _GSUB, _GSUB), :], sem)
                for s in range((_CD * _K) // _GSUB)
            ]
            pltpu.sync_copy(upre_hbm.at[pl.ds(base, _CD), :], u_v)
            for cp in copies:
                cp.wait()

            def one_row(r, _):
                for q in range(dout // 16):
                    fs = pl.ds(q * 16, 16)
                    acc = rows_v[r * _K, fs]
                    for t in range(1, _K):
                        acc = jnp.maximum(acc, rows_v[r * _K + t, fs])
                    out_v[r, fs] = jnp.maximum(acc + u_v[r, fs], 0.0)
                return _

            lax.fori_loop(0, _CD, one_row, 0, unroll=False)
            pltpu.sync_copy(out_v, out_hbm.at[pl.ds(base, _CD), :])
            return _

        lax.fori_loop(0, nchunk, chunk, 0, unroll=False)

    return k(v, upre, idx_flat)


def _head_body(refs):
    xf_ref, wf1_ref, bf1_ref, wf2_ref, bf2_ref, out_ref = refs
    h = jnp.maximum(jnp.dot(xf_ref[...], wf1_ref[...],
                            preferred_element_type=jnp.float32)
                    + bf1_ref[...], 0.0)
    out_ref[...] = jnp.dot(h, wf2_ref[...],
                           preferred_element_type=jnp.float32) + bf2_ref[...]


def _head(xf, wf1, bf1, wf2, bf2):
    m, d = xf.shape
    full = lambda s: pl.BlockSpec(s, lambda: (0, 0))
    return pl.pallas_call(
        lambda *refs: _head_body(refs),
        in_specs=[full((m, d)), full(wf1.shape), full((1, wf1.shape[1])),
                  full(wf2.shape), full((1, wf2.shape[1]))],
        out_specs=full((m, wf2.shape[1])),
        out_shape=jax.ShapeDtypeStruct((m, wf2.shape[1]), jnp.float32),
    )(xf, wf1, bf1.reshape(1, -1), wf2, bf2.reshape(1, -1))


def kernel(x, W1, b1, W2, b2, W3, b3, Wf1, bf1, Wf2, bf2):
    B, N, _ = x.shape
    xf = x.reshape(B * N, 3)
    for w, b in ((W1, b1), (W2, b2), (W3, b3)):
        idx, upre, v = _tc_layer(xf, w, b)
        xf = _sc_gather_max(v, upre, idx.reshape(-1))
    out = _head(xf, Wf1, bf1, Wf2, bf2)
    return out.reshape(B, N, 1)
